# baseline (device time: 465183 ns/iter reference)
import jax
import jax.numpy as jnp
from jax import lax
from jax.experimental import pallas as pl
from jax.experimental.pallas import tpu as pltpu

NZ = 4
M, N = 16384, 1024
QR = M // 4
CR = QR // NZ
NSUB = 4
SC = CR // NSUB
NSLOT = 4


def kernel(x):
    def body(x_ref, out_ref, qcomm_ref, qacc_ref, qxc_ref,
             z_send, z_recv, ag_send, ag_recv, xy_send, xy_recv,
             p2_send, p2_recv, copy_sems):
        my_x = lax.axis_index("x")
        my_y = lax.axis_index("y")
        my_z = lax.axis_index("z")
        zright = (my_z + 1) % NZ
        zleft = (my_z + NZ - 1) % NZ

        q = my_x * 2 + my_y
        qx = (1 - my_x) * 2 + my_y
        qy = my_x * 2 + (1 - my_y)
        qd = (1 - my_x) * 2 + (1 - my_y)
        q0 = q * QR

        barrier = pltpu.get_barrier_semaphore()
        for dev in ((my_x, my_y, zleft), (my_x, my_y, zright),
                    (1 - my_x, my_y, my_z), (my_x, 1 - my_y, my_z)):
            pl.semaphore_signal(barrier, inc=1, device_id=dev,
                                device_id_type=pl.DeviceIdType.MESH)
        pl.semaphore_wait(barrier, 4)

        cp = pltpu.make_async_copy(
            x_ref.at[pl.ds(q0 + my_z * CR, CR), :], qacc_ref,
            copy_sems.at[0])
        cp.start()
        cp.wait()

        def rs_desc(h, s):
            return pltpu.make_async_remote_copy(
                src_ref=qacc_ref.at[pl.ds(s * SC, SC), :],
                dst_ref=qcomm_ref.at[h, s],
                send_sem=z_send.at[h * NSUB + s],
                recv_sem=z_recv.at[h * NSUB + s],
                device_id=(my_x, my_y, zright),
                device_id_type=pl.DeviceIdType.MESH,
            )

        rz = (my_z + 1) % NZ
        xdev = (1 - my_x, my_y, my_z)
        ydev = (my_x, 1 - my_y, my_z)
        Hs = CR // 2
        p1 = {}

        def xy_half(half):
            for li, dev in enumerate((xdev, ydev)):
                si = 2 * half + li
                xy = pltpu.make_async_remote_copy(
                    src_ref=qacc_ref.at[pl.ds(half * Hs, Hs), :],
                    dst_ref=out_ref.at[pl.ds(q0 + rz * CR + half * Hs, Hs), :],
                    send_sem=xy_send.at[si],
                    recv_sem=xy_recv.at[si],
                    device_id=dev,
                    device_id_type=pl.DeviceIdType.MESH,
                )
                xy.start()
                p1.setdefault((0, li), []).append(xy)

        desc = {}
        for s in range(NSUB):
            desc[(0, s)] = rs_desc(0, s)
            desc[(0, s)].start()
        cpx = pltpu.make_async_copy(
            x_ref.at[pl.ds(q0 + ((my_z - 1) % NZ) * CR, CR), :], qxc_ref,
            copy_sems.at[1])
        cpx.start()
        for h in range(NZ - 1):
            cpx.wait()
            for s in range(NSUB):
                desc[(h, s)].wait()
                qacc_ref[pl.ds(s * SC, SC), :] = (
                    qcomm_ref[h, s] + qxc_ref[pl.ds(s * SC, SC), :])
                if h < NZ - 2:
                    desc[(h + 1, s)] = rs_desc(h + 1, s)
                    desc[(h + 1, s)].start()
                elif s == 1:
                    xy_half(0)
                elif s == NSUB - 1:
                    xy_half(1)
            if h < NZ - 2:
                rc = (my_z - h - 2) % NZ
                cpx = pltpu.make_async_copy(
                    x_ref.at[pl.ds(q0 + rc * CR, CR), :], qxc_ref,
                    copy_sems.at[h % 2])
                cpx.start()

        cpo = pltpu.make_async_copy(
            qacc_ref, out_ref.at[pl.ds(q0 + rz * CR, CR), :],
            copy_sems.at[0])
        cpo.start()
        cpo.wait()

        Hh = CR // 2
        relays = []

        def relay(j):
            cj = (my_z + 1 - j) % NZ
            for d in p1[(j, 0)]:
                d.wait_recv()
            for d in p1[(j, 1)]:
                d.wait_recv()
            xrow = qy * QR + cj * CR
            yrow = qx * QR + cj * CR + Hh
            for li, (dev, row) in enumerate(((xdev, xrow), (ydev, yrow))):
                si = 2 * j + li
                rl = pltpu.make_async_remote_copy(
                    src_ref=out_ref.at[pl.ds(row, Hh), :],
                    dst_ref=out_ref.at[pl.ds(row, Hh), :],
                    send_sem=p2_send.at[si],
                    recv_sem=p2_recv.at[si],
                    device_id=dev,
                    device_id_type=pl.DeviceIdType.MESH,
                )
                rl.start()
                relays.append(rl)

        for r in range(NZ):
            cr = (my_z + 1 - r) % NZ
            row = q0 + cr * CR
            ag = None
            if r < NZ - 1:
                src = (qacc_ref if r == 0
                       else out_ref.at[pl.ds(row, CR), :])
                ag = pltpu.make_async_remote_copy(
                    src_ref=src,
                    dst_ref=out_ref.at[pl.ds(row, CR), :],
                    send_sem=ag_send.at[r],
                    recv_sem=ag_recv.at[r],
                    device_id=(my_x, my_y, zright),
                    device_id_type=pl.DeviceIdType.MESH,
                )
                ag.start()
            if r > 0:
                for li, dev in enumerate((xdev, ydev)):
                    si = 2 + 2 * r + li
                    xy = pltpu.make_async_remote_copy(
                        src_ref=out_ref.at[pl.ds(row, CR), :],
                        dst_ref=out_ref.at[pl.ds(row, CR), :],
                        send_sem=xy_send.at[si],
                        recv_sem=xy_recv.at[si],
                        device_id=dev,
                        device_id_type=pl.DeviceIdType.MESH,
                    )
                    xy.start()
                    p1[(r, li)] = [xy]
            if r > 0:
                relay(r - 1)
            if ag is not None:
                ag.wait()

        relay(NZ - 1)

        for descs in p1.values():
            for xy in descs:
                xy.wait_send()
        for rl in relays:
            rl.wait()

    return pl.pallas_call(
        body,
        out_shape=jax.ShapeDtypeStruct((M, N), jnp.float32),
        in_specs=[pl.BlockSpec(memory_space=pl.ANY)],
        out_specs=pl.BlockSpec(memory_space=pl.ANY),
        scratch_shapes=[
            pltpu.VMEM((NZ - 1, NSUB, SC, N), jnp.float32),
            pltpu.VMEM((CR, N), jnp.float32),
            pltpu.VMEM((CR, N), jnp.float32),
            pltpu.SemaphoreType.DMA(((NZ - 1) * NSUB,)),
            pltpu.SemaphoreType.DMA(((NZ - 1) * NSUB,)),
            pltpu.SemaphoreType.DMA((NZ - 1,)),
            pltpu.SemaphoreType.DMA((NZ - 1,)),
            pltpu.SemaphoreType.DMA((10,)),
            pltpu.SemaphoreType.DMA((10,)),
            pltpu.SemaphoreType.DMA((8,)),
            pltpu.SemaphoreType.DMA((8,)),
            pltpu.SemaphoreType.DMA((2,)),
        ],
        compiler_params=pltpu.CompilerParams(collective_id=0),
    )(x)


# device time: 439476 ns/iter; 1.0585x vs baseline; 1.0585x over previous
import jax
import jax.numpy as jnp
from jax import lax
from jax.experimental import pallas as pl
from jax.experimental.pallas import tpu as pltpu

NZ = 4
M, N = 16384, 1024
QR = M // 4
CR = QR // NZ
NSUB = 4
SC = CR // NSUB
NSLOT = 4


def kernel(x):
    def body(x_ref, out_ref, qcomm_ref, qacc_ref, qxc_ref,
             z_send, z_recv, ag_send, ag_recv, xy_send, xy_recv,
             p2_send, p2_recv, copy_sems):
        my_x = lax.axis_index("x")
        my_y = lax.axis_index("y")
        my_z = lax.axis_index("z")
        zright = (my_z + 1) % NZ
        zleft = (my_z + NZ - 1) % NZ

        q = my_x * 2 + my_y
        qx = (1 - my_x) * 2 + my_y
        qy = my_x * 2 + (1 - my_y)
        qd = (1 - my_x) * 2 + (1 - my_y)
        q0 = q * QR

        barrier = pltpu.get_barrier_semaphore()
        for dev in ((my_x, my_y, zleft), (my_x, my_y, zright),
                    (1 - my_x, my_y, my_z), (my_x, 1 - my_y, my_z)):
            pl.semaphore_signal(barrier, inc=1, device_id=dev,
                                device_id_type=pl.DeviceIdType.MESH)
        pl.semaphore_wait(barrier, 4)

        cp = pltpu.make_async_copy(
            x_ref.at[pl.ds(q0 + my_z * CR, CR), :], qacc_ref,
            copy_sems.at[0])
        cp.start()
        cp.wait()

        def rs_desc(h, s):
            return pltpu.make_async_remote_copy(
                src_ref=qacc_ref.at[pl.ds(s * SC, SC), :],
                dst_ref=qcomm_ref.at[h, s],
                send_sem=z_send.at[h * NSUB + s],
                recv_sem=z_recv.at[h * NSUB + s],
                device_id=(my_x, my_y, zright),
                device_id_type=pl.DeviceIdType.MESH,
            )

        rz = (my_z + 1) % NZ
        xdev = (1 - my_x, my_y, my_z)
        ydev = (my_x, 1 - my_y, my_z)
        Hs = CR // 2
        p1 = {}

        ag_desc = {}

        def xy_half(half):
            for li, dev in enumerate((xdev, ydev)):
                si = 2 * half + li
                xy = pltpu.make_async_remote_copy(
                    src_ref=qacc_ref.at[pl.ds(half * Hs, Hs), :],
                    dst_ref=out_ref.at[pl.ds(q0 + rz * CR + half * Hs, Hs), :],
                    send_sem=xy_send.at[si],
                    recv_sem=xy_recv.at[si],
                    device_id=dev,
                    device_id_type=pl.DeviceIdType.MESH,
                )
                xy.start()
                p1.setdefault((0, li), []).append(xy)
            ag = pltpu.make_async_remote_copy(
                src_ref=qacc_ref.at[pl.ds(half * Hs, Hs), :],
                dst_ref=out_ref.at[pl.ds(q0 + rz * CR + half * Hs, Hs), :],
                send_sem=ag_send.at[half],
                recv_sem=ag_recv.at[half],
                device_id=(my_x, my_y, zright),
                device_id_type=pl.DeviceIdType.MESH,
            )
            ag.start()
            ag_desc[(0, half)] = ag

        desc = {}
        for s in range(NSUB):
            desc[(0, s)] = rs_desc(0, s)
            desc[(0, s)].start()
        cpx = pltpu.make_async_copy(
            x_ref.at[pl.ds(q0 + ((my_z - 1) % NZ) * CR, CR), :], qxc_ref,
            copy_sems.at[1])
        cpx.start()
        for h in range(NZ - 1):
            cpx.wait()
            for s in range(NSUB):
                desc[(h, s)].wait()
                qacc_ref[pl.ds(s * SC, SC), :] = (
                    qcomm_ref[h, s] + qxc_ref[pl.ds(s * SC, SC), :])
                if h < NZ - 2:
                    desc[(h + 1, s)] = rs_desc(h + 1, s)
                    desc[(h + 1, s)].start()
                elif s == 1:
                    xy_half(0)
                elif s == NSUB - 1:
                    xy_half(1)
            if h < NZ - 2:
                rc = (my_z - h - 2) % NZ
                cpx = pltpu.make_async_copy(
                    x_ref.at[pl.ds(q0 + rc * CR, CR), :], qxc_ref,
                    copy_sems.at[h % 2])
                cpx.start()

        cpo = pltpu.make_async_copy(
            qacc_ref, out_ref.at[pl.ds(q0 + rz * CR, CR), :],
            copy_sems.at[0])
        cpo.start()
        cpo.wait()

        Hh = CR // 2
        relays = []

        def relay(j):
            cj = (my_z + 1 - j) % NZ
            for d in p1[(j, 0)]:
                d.wait_recv()
            for d in p1[(j, 1)]:
                d.wait_recv()
            xrow = qy * QR + cj * CR
            yrow = qx * QR + cj * CR + Hh
            for li, (dev, row) in enumerate(((xdev, xrow), (ydev, yrow))):
                si = 2 * j + li
                rl = pltpu.make_async_remote_copy(
                    src_ref=out_ref.at[pl.ds(row, Hh), :],
                    dst_ref=out_ref.at[pl.ds(row, Hh), :],
                    send_sem=p2_send.at[si],
                    recv_sem=p2_recv.at[si],
                    device_id=dev,
                    device_id_type=pl.DeviceIdType.MESH,
                )
                rl.start()
                relays.append(rl)

        for r in range(1, NZ):
            cr = (my_z + 1 - r) % NZ
            row = q0 + cr * CR
            for hh in range(2):
                ag_desc[(r - 1, hh)].wait()
                hrow = row + hh * Hs
                if r < NZ - 1:
                    ag = pltpu.make_async_remote_copy(
                        src_ref=out_ref.at[pl.ds(hrow, Hs), :],
                        dst_ref=out_ref.at[pl.ds(hrow, Hs), :],
                        send_sem=ag_send.at[2 * r + hh],
                        recv_sem=ag_recv.at[2 * r + hh],
                        device_id=(my_x, my_y, zright),
                        device_id_type=pl.DeviceIdType.MESH,
                    )
                    ag.start()
                    ag_desc[(r, hh)] = ag
                for li, dev in enumerate((xdev, ydev)):
                    si = 4 * r + 2 * hh + li
                    xy = pltpu.make_async_remote_copy(
                        src_ref=out_ref.at[pl.ds(hrow, Hs), :],
                        dst_ref=out_ref.at[pl.ds(hrow, Hs), :],
                        send_sem=xy_send.at[si],
                        recv_sem=xy_recv.at[si],
                        device_id=dev,
                        device_id_type=pl.DeviceIdType.MESH,
                    )
                    xy.start()
                    p1.setdefault((r, li), []).append(xy)
            relay(r - 1)

        relay(NZ - 1)

        for descs in p1.values():
            for xy in descs:
                xy.wait_send()
        for rl in relays:
            rl.wait()

    return pl.pallas_call(
        body,
        out_shape=jax.ShapeDtypeStruct((M, N), jnp.float32),
        in_specs=[pl.BlockSpec(memory_space=pl.ANY)],
        out_specs=pl.BlockSpec(memory_space=pl.ANY),
        scratch_shapes=[
            pltpu.VMEM((NZ - 1, NSUB, SC, N), jnp.float32),
            pltpu.VMEM((CR, N), jnp.float32),
            pltpu.VMEM((CR, N), jnp.float32),
            pltpu.SemaphoreType.DMA(((NZ - 1) * NSUB,)),
            pltpu.SemaphoreType.DMA(((NZ - 1) * NSUB,)),
            pltpu.SemaphoreType.DMA((6,)),
            pltpu.SemaphoreType.DMA((6,)),
            pltpu.SemaphoreType.DMA((16,)),
            pltpu.SemaphoreType.DMA((16,)),
            pltpu.SemaphoreType.DMA((8,)),
            pltpu.SemaphoreType.DMA((8,)),
            pltpu.SemaphoreType.DMA((2,)),
        ],
        compiler_params=pltpu.CompilerParams(collective_id=0),
    )(x)
